# baseline (speedup 1.0000x reference)
import functools

import jax
import jax.numpy as jnp
from jax import lax
from jax.experimental import pallas as pl
from jax.experimental.pallas import tpu as pltpu
from jax.experimental.pallas import tpu_sc as plsc

N = 10000
D = 256
DH = D // 2
E = 160000
NC = 2
NS = 16
EP = E // NS
PIECE = 2000
HALF = N // NC
QLEN = (2504, 2496)
ACC = 2560
B = 80
CAP = EP + 2 * B + 16
OUT_T = 160


def _sc_agg_body(xa_hbm, xb_hbm, src_hbm, dst_hbm, suma_hbm, sumb_hbm, cnt_hbm,
                 src_p, dst_p, csrc_v, cdst_v,
                 rows0a_v, rows0b_v, rows1a_v, rows1b_v, ones_v,
                 sidx0_v, sidx1_v, didx0_v, didx1_v,
                 acca_sh, accb_sh, cacc_sh,
                 sem0a, sem0b, sem1a, sem1b, *, with_cnt):
    c = lax.axis_index("c")
    s = lax.axis_index("s")
    wid = s * NC + c
    lane = lax.iota(jnp.int32, 16)
    zeros16 = jnp.zeros((16,), jnp.float32)
    ones16 = jnp.ones((16,), jnp.float32)

    def _zrow(r, carry):
        for k in range(DH // 16):
            rows0a_v[r, pl.ds(k * 16, 16)] = zeros16
        ones_v[r, pl.ds(0, 16)] = zeros16
        return carry
    lax.fori_loop(0, B, _zrow, 0)

    for q in range(2):
        qlen = QLEN[q]
        base_row = c * HALF + q * QLEN[0]

        r0 = s * (ACC // NS)
        for acc in (acca_sh, accb_sh):
            pltpu.sync_copy(rows0a_v, acc.at[pl.ds(r0, B)])
            pltpu.sync_copy(rows0a_v, acc.at[pl.ds(r0 + B, B)])
        if with_cnt:
            pltpu.sync_copy(ones_v, cacc_sh.at[pl.ds(r0, B)])
            pltpu.sync_copy(ones_v, cacc_sh.at[pl.ds(r0 + B, B)])

            def _fone(r, carry):
                ones_v[r, pl.ds(0, 16)] = ones16
                return carry
            lax.fori_loop(0, B, _fone, 0)

        def _piece(p, nacc):
            pltpu.sync_copy(src_hbm.at[pl.ds(s * EP + p * PIECE, PIECE)], src_p)
            pltpu.sync_copy(dst_hbm.at[pl.ds(s * EP + p * PIECE, PIECE)], dst_p)

            def _fstep(o, base):
                dvec = dst_p[pl.ds(o, 16)]
                svec = src_p[pl.ds(o, 16)]
                loc = dvec - base_row
                m = (loc >= 0) & (loc < qlen)
                mi = m.astype(jnp.int32)
                pos = base + plsc.cumsum(mi) - mi
                plsc.store_scatter(cdst_v, [pos], loc, mask=m)
                plsc.store_scatter(csrc_v, [pos], svec, mask=m)
                return base + plsc.all_reduce_population_count(m)

            def _fbody(i, nacc_vec):
                base = _fstep(i * 32, nacc_vec)
                return _fstep(i * 32 + 16, base)

            base = lax.fori_loop(0, PIECE // 32, _fbody, nacc)
            return _fstep((PIECE // 32) * 32, base)

        nacc_vec = lax.fori_loop(0, EP // PIECE, _piece,
                                 jnp.zeros((16,), jnp.int32))
        n = jnp.max(nacc_vec)

        pad_src = wid * 16 + lane
        pad_dst = QLEN[0] + ((wid * 16 + lane) % (ACC - QLEN[0]))
        for k in range(2 * B // 16):
            plsc.store_scatter(cdst_v, [n + k * 16 + lane], pad_dst)
            plsc.store_scatter(csrc_v, [n + k * 16 + lane], pad_src)

        plsc.subcore_barrier()

        def _stage_start(b, sidx_v, didx_v, rows_a, rows_b, sem_a, sem_b):
            off = b * B
            for k in range(B // 16):
                sidx_v[pl.ds(k * 16, 16)] = csrc_v[pl.ds(off + k * 16, 16)]
                didx_v[pl.ds(k * 16, 16)] = cdst_v[pl.ds(off + k * 16, 16)]
            pltpu.async_copy(xa_hbm.at[sidx_v], rows_a, sem_a)
            pltpu.async_copy(xb_hbm.at[sidx_v], rows_b, sem_b)

        def _wait(sidx_v, rows_a, rows_b, sem_a, sem_b):
            pltpu.make_async_copy(xa_hbm.at[sidx_v], rows_a, sem_a).wait()
            pltpu.make_async_copy(xb_hbm.at[sidx_v], rows_b, sem_b).wait()

        def _scatter(didx_v, rows_a, rows_b):
            pltpu.sync_copy(rows_a, acca_sh.at[didx_v], add=True)
            pltpu.sync_copy(rows_b, accb_sh.at[didx_v], add=True)
            if with_cnt:
                pltpu.sync_copy(ones_v, cacc_sh.at[didx_v], add=True)

        npairs = jnp.maximum((n + 2 * B - 1) // (2 * B), 1)
        _stage_start(jnp.int32(0), sidx0_v, didx0_v, rows0a_v, rows0b_v,
                     sem0a, sem0b)

        def _pbody(p, carry):
            _stage_start(2 * p + 1, sidx1_v, didx1_v, rows1a_v, rows1b_v,
                         sem1a, sem1b)
            _wait(sidx0_v, rows0a_v, rows0b_v, sem0a, sem0b)
            _scatter(didx0_v, rows0a_v, rows0b_v)

            @pl.when(p + 1 < npairs)
            def _():
                _stage_start(2 * p + 2, sidx0_v, didx0_v, rows0a_v, rows0b_v,
                             sem0a, sem0b)
            _wait(sidx1_v, rows1a_v, rows1b_v, sem1a, sem1b)
            _scatter(didx1_v, rows1a_v, rows1b_v)
            return carry

        lax.fori_loop(0, npairs, _pbody, 0)

        plsc.subcore_barrier()

        rem = qlen - (NS - 1) * OUT_T

        @pl.when(s < NS - 1)
        def _():
            rb = s * OUT_T
            pltpu.sync_copy(acca_sh.at[pl.ds(rb, OUT_T)],
                            suma_hbm.at[pl.ds(base_row + rb, OUT_T)])
            pltpu.sync_copy(accb_sh.at[pl.ds(rb, OUT_T)],
                            sumb_hbm.at[pl.ds(base_row + rb, OUT_T)])
            if with_cnt:
                pltpu.sync_copy(cacc_sh.at[pl.ds(rb, OUT_T)],
                                cnt_hbm.at[pl.ds(base_row + rb, OUT_T)])

        @pl.when(s == NS - 1)
        def _():
            rb = (NS - 1) * OUT_T
            pltpu.sync_copy(acca_sh.at[pl.ds(rb, rem)],
                            suma_hbm.at[pl.ds(base_row + rb, rem)])
            pltpu.sync_copy(accb_sh.at[pl.ds(rb, rem)],
                            sumb_hbm.at[pl.ds(base_row + rb, rem)])
            if with_cnt:
                pltpu.sync_copy(cacc_sh.at[pl.ds(rb, rem)],
                                cnt_hbm.at[pl.ds(base_row + rb, rem)])

        if q == 0:
            lax.fori_loop(0, B, _zrow, 0)
            plsc.subcore_barrier()


def _make_sc_agg(with_cnt):
    body = functools.partial(_sc_agg_body, with_cnt=with_cnt)
    if with_cnt:
        def wrapped(xa, xb, src, dst, suma, sumb, cnt, *rest):
            body(xa, xb, src, dst, suma, sumb, cnt, *rest)
        out_type = (jax.ShapeDtypeStruct((N, DH), jnp.float32),
                    jax.ShapeDtypeStruct((N, DH), jnp.float32),
                    jax.ShapeDtypeStruct((N, 16), jnp.float32))
    else:
        def wrapped(xa, xb, src, dst, suma, sumb, *rest):
            body(xa, xb, src, dst, suma, sumb, None, *rest)
        out_type = (jax.ShapeDtypeStruct((N, DH), jnp.float32),
                    jax.ShapeDtypeStruct((N, DH), jnp.float32))
    return functools.partial(
        pl.kernel,
        mesh=plsc.VectorSubcoreMesh(core_axis_name="c", subcore_axis_name="s",
                                    num_cores=NC),
        compiler_params=pltpu.CompilerParams(needs_layout_passes=False,
                                             use_tc_tiling_on_sc=False),
        out_type=out_type,
        scratch_types=[
            pltpu.VMEM((PIECE,), jnp.int32),
            pltpu.VMEM((PIECE,), jnp.int32),
            pltpu.VMEM((CAP,), jnp.int32),
            pltpu.VMEM((CAP,), jnp.int32),
            pltpu.VMEM((B, DH), jnp.float32),
            pltpu.VMEM((B, DH), jnp.float32),
            pltpu.VMEM((B, DH), jnp.float32),
            pltpu.VMEM((B, DH), jnp.float32),
            pltpu.VMEM((B, 16), jnp.float32),
            pltpu.VMEM((B,), jnp.int32),
            pltpu.VMEM((B,), jnp.int32),
            pltpu.VMEM((B,), jnp.int32),
            pltpu.VMEM((B,), jnp.int32),
            pltpu.VMEM_SHARED((ACC, DH), jnp.float32),
            pltpu.VMEM_SHARED((ACC, DH), jnp.float32),
            pltpu.VMEM_SHARED((ACC, 16), jnp.float32),
            pltpu.SemaphoreType.DMA,
            pltpu.SemaphoreType.DMA,
            pltpu.SemaphoreType.DMA,
            pltpu.SemaphoreType.DMA,
        ],
    )(wrapped)


_sc_agg_cnt = _make_sc_agg(with_cnt=True)
_sc_agg_nocnt = _make_sc_agg(with_cnt=False)


RB = 1000


def _self_body(xa_ref, xb_ref, wr_ref, b_ref, o_ref):
    acc = lax.dot_general(xa_ref[...], wr_ref[:, 0:DH],
                          (((1,), (1,)), ((), ())),
                          preferred_element_type=jnp.float32)
    acc = acc + lax.dot_general(xb_ref[...], wr_ref[:, DH:D],
                                (((1,), (1,)), ((), ())),
                                preferred_element_type=jnp.float32)
    o_ref[...] = acc + b_ref[...]


def _self(xa, xb, Wr, b):
    return pl.pallas_call(
        _self_body,
        grid=(N // RB,),
        in_specs=[
            pl.BlockSpec((RB, DH), lambda i: (i, 0)),
            pl.BlockSpec((RB, DH), lambda i: (i, 0)),
            pl.BlockSpec((D, D), lambda i: (0, 0)),
            pl.BlockSpec((1, D), lambda i: (0, 0)),
        ],
        out_specs=pl.BlockSpec((RB, D), lambda i: (i, 0)),
        out_shape=jax.ShapeDtypeStruct((N, D), jnp.float32),
    )(xa, xb, Wr, b.reshape(1, D))


def _combine_body(suma_ref, sumb_ref, cnt_ref, t_ref, wl_ref,
                  oa_ref, ob_ref, *, relu, split_out):
    cnt = cnt_ref[:, 0:1]
    inv = 1.0 / jnp.maximum(cnt, 1.0)
    acc = lax.dot_general(suma_ref[...] * inv, wl_ref[:, 0:DH],
                          (((1,), (1,)), ((), ())),
                          preferred_element_type=jnp.float32)
    acc = acc + lax.dot_general(sumb_ref[...] * inv, wl_ref[:, DH:D],
                                (((1,), (1,)), ((), ())),
                                preferred_element_type=jnp.float32)
    acc = acc + t_ref[...]
    if relu:
        acc = jnp.maximum(acc, 0.0)
    if split_out:
        oa_ref[...] = acc[:, 0:DH]
        ob_ref[...] = acc[:, DH:D]
    else:
        oa_ref[...] = acc


def _combine(suma, sumb, cnt16, t, Wl, relu, split_out):
    if split_out:
        out_specs = (pl.BlockSpec((RB, DH), lambda i: (i, 0)),
                     pl.BlockSpec((RB, DH), lambda i: (i, 0)))
        out_shape = (jax.ShapeDtypeStruct((N, DH), jnp.float32),
                     jax.ShapeDtypeStruct((N, DH), jnp.float32))
    else:
        out_specs = pl.BlockSpec((RB, D), lambda i: (i, 0))
        out_shape = jax.ShapeDtypeStruct((N, D), jnp.float32)
    body = functools.partial(_combine_body, relu=relu, split_out=split_out)
    if not split_out:
        body_wrapped = lambda *refs: body(*refs[:-1], refs[-1], None)

        def body_wrapped(suma_ref, sumb_ref, cnt_ref, t_ref, wl_ref, o_ref):
            body(suma_ref, sumb_ref, cnt_ref, t_ref, wl_ref, o_ref, None)
    else:
        body_wrapped = body
    return pl.pallas_call(
        body_wrapped,
        grid=(N // RB,),
        in_specs=[
            pl.BlockSpec((RB, DH), lambda i: (i, 0)),
            pl.BlockSpec((RB, DH), lambda i: (i, 0)),
            pl.BlockSpec((RB, 16), lambda i: (i, 0)),
            pl.BlockSpec((RB, D), lambda i: (i, 0)),
            pl.BlockSpec((D, D), lambda i: (0, 0)),
        ],
        out_specs=out_specs,
        out_shape=out_shape,
    )(suma, sumb, cnt16, t, Wl)


def kernel(x, edge_index, W1l, b1l, W1r, W2l, b2l, W2r):
    ei = edge_index.astype(jnp.int32)
    src = ei[0]
    dst = ei[1]
    xa = x[:, 0:DH]
    xb = x[:, DH:D]
    s1a, s1b, c1 = _sc_agg_cnt(xa, xb, src, dst)
    t1 = _self(xa, xb, W1r, b1l)
    ha, hb = _combine(s1a, s1b, c1, t1, W1l, relu=True, split_out=True)
    s2a, s2b = _sc_agg_nocnt(ha, hb, src, dst)
    t2 = _self(ha, hb, W2r, b2l)
    out = _combine(s2a, s2b, c1, t2, W2l, relu=False, split_out=False)
    return out


# speedup vs baseline: 1.0911x; 1.0911x over previous
import functools

import jax
import jax.numpy as jnp
from jax import lax
from jax.experimental import pallas as pl
from jax.experimental.pallas import tpu as pltpu
from jax.experimental.pallas import tpu_sc as plsc

N = 10000
D = 256
DH = D // 2
E = 160000
NC = 2
NS = 16
EP = E // NS
PIECE = 2000
HALF = N // NC
QLEN = (2504, 2496)
ACC = 2560
B = 80
CAP = EP + 4 * B + 32
OUT_T = 160


def _sc_agg_body(xa_hbm, xb_hbm, src_hbm, dst_hbm, suma_hbm, sumb_hbm, cnt_hbm,
                 src_p, dst_p, csrc_v, cdst_v,
                 rows0a_v, rows0b_v, rows1a_v, rows1b_v, ones_v,
                 sidx0_v, sidx1_v, didx0_v, didx1_v,
                 acca_sh, accb_sh, cacc_sh,
                 sem0a, sem0b, sem1a, sem1b, *, with_cnt):
    c = lax.axis_index("c")
    s = lax.axis_index("s")
    wid = s * NC + c
    lane = lax.iota(jnp.int32, 16)
    zeros16 = jnp.zeros((16,), jnp.float32)
    ones16 = jnp.ones((16,), jnp.float32)

    def _zrow(r, carry):
        for k in range(DH // 16):
            rows0a_v[r, pl.ds(k * 16, 16)] = zeros16
        ones_v[r, pl.ds(0, 16)] = zeros16
        return carry
    lax.fori_loop(0, B, _zrow, 0)

    cbase = c * HALF

    def _piece(p, carry):
        pltpu.sync_copy(src_hbm.at[pl.ds(s * EP + p * PIECE, PIECE)], src_p)
        pltpu.sync_copy(dst_hbm.at[pl.ds(s * EP + p * PIECE, PIECE)], dst_p)

        def _fstep(o, b0, b1):
            dvec = dst_p[pl.ds(o, 16)]
            svec = src_p[pl.ds(o, 16)]
            loc = dvec - cbase
            m0 = (loc >= 0) & (loc < QLEN[0])
            loc1 = loc - QLEN[0]
            m1 = (loc1 >= 0) & (loc1 < QLEN[1])
            m0i = m0.astype(jnp.int32)
            m1i = m1.astype(jnp.int32)
            pos0 = b0 + plsc.cumsum(m0i) - m0i
            plsc.store_scatter(cdst_v, [pos0], loc, mask=m0)
            plsc.store_scatter(csrc_v, [pos0], svec, mask=m0)
            pos1 = (CAP - 1) - (b1 + plsc.cumsum(m1i) - m1i)
            plsc.store_scatter(cdst_v, [pos1], loc1, mask=m1)
            plsc.store_scatter(csrc_v, [pos1], svec, mask=m1)
            return b0 + plsc.all_reduce_population_count(m0), \
                   b1 + plsc.all_reduce_population_count(m1)

        def _fbody(i, carry):
            return _fstep(i * 16, *carry)

        return lax.fori_loop(0, PIECE // 16, _fbody, carry)

    zero_i32 = jnp.zeros((16,), jnp.int32)
    n0_vec, n1_vec = lax.fori_loop(0, EP // PIECE, _piece,
                                   (zero_i32, zero_i32))
    n0 = jnp.max(n0_vec)
    n1 = jnp.max(n1_vec)

    pad_src = wid * 16 + lane
    pad_dst = QLEN[0] + ((wid * 16 + lane) % (ACC - QLEN[0]))
    for k in range(2 * B // 16):
        plsc.store_scatter(cdst_v, [n0 + k * 16 + lane], pad_dst)
        plsc.store_scatter(csrc_v, [n0 + k * 16 + lane], pad_src)
        plsc.store_scatter(cdst_v, [(CAP - 1) - (n1 + k * 16 + lane)], pad_dst)
        plsc.store_scatter(csrc_v, [(CAP - 1) - (n1 + k * 16 + lane)], pad_src)

    npairs0 = jnp.maximum((n0 + 2 * B - 1) // (2 * B), 1)
    npairs1 = jnp.maximum((n1 + 2 * B - 1) // (2 * B), 1)

    for q in range(2):
        qlen = QLEN[q]
        base_row = c * HALF + q * QLEN[0]
        npairs = npairs0 if q == 0 else npairs1
        list_base = jnp.int32(0) if q == 0 else CAP - npairs1 * 2 * B

        r0 = s * (ACC // NS)
        for acc in (acca_sh, accb_sh):
            pltpu.sync_copy(rows0a_v, acc.at[pl.ds(r0, B)])
            pltpu.sync_copy(rows0a_v, acc.at[pl.ds(r0 + B, B)])
        if with_cnt:
            pltpu.sync_copy(ones_v, cacc_sh.at[pl.ds(r0, B)])
            pltpu.sync_copy(ones_v, cacc_sh.at[pl.ds(r0 + B, B)])

            def _fone(r, carry):
                ones_v[r, pl.ds(0, 16)] = ones16
                return carry
            lax.fori_loop(0, B, _fone, 0)

        plsc.subcore_barrier()

        def _stage_start(b, sidx_v, didx_v, rows_a, rows_b, sem_a, sem_b):
            off = list_base + b * B
            for k in range(B // 16):
                sidx_v[pl.ds(k * 16, 16)] = csrc_v[pl.ds(off + k * 16, 16)]
                didx_v[pl.ds(k * 16, 16)] = cdst_v[pl.ds(off + k * 16, 16)]
            pltpu.async_copy(xa_hbm.at[sidx_v], rows_a, sem_a)
            pltpu.async_copy(xb_hbm.at[sidx_v], rows_b, sem_b)

        def _wait(sidx_v, rows_a, rows_b, sem_a, sem_b):
            pltpu.make_async_copy(xa_hbm.at[sidx_v], rows_a, sem_a).wait()
            pltpu.make_async_copy(xb_hbm.at[sidx_v], rows_b, sem_b).wait()

        def _scatter(didx_v, rows_a, rows_b):
            pltpu.sync_copy(rows_a, acca_sh.at[didx_v], add=True)
            pltpu.sync_copy(rows_b, accb_sh.at[didx_v], add=True)
            if with_cnt:
                pltpu.sync_copy(ones_v, cacc_sh.at[didx_v], add=True)

        _stage_start(jnp.int32(0), sidx0_v, didx0_v, rows0a_v, rows0b_v,
                     sem0a, sem0b)

        def _pbody(p, carry):
            _stage_start(2 * p + 1, sidx1_v, didx1_v, rows1a_v, rows1b_v,
                         sem1a, sem1b)
            _wait(sidx0_v, rows0a_v, rows0b_v, sem0a, sem0b)
            _scatter(didx0_v, rows0a_v, rows0b_v)

            @pl.when(p + 1 < npairs)
            def _():
                _stage_start(2 * p + 2, sidx0_v, didx0_v, rows0a_v, rows0b_v,
                             sem0a, sem0b)
            _wait(sidx1_v, rows1a_v, rows1b_v, sem1a, sem1b)
            _scatter(didx1_v, rows1a_v, rows1b_v)
            return carry

        lax.fori_loop(0, npairs, _pbody, 0)

        plsc.subcore_barrier()

        rem = qlen - (NS - 1) * OUT_T

        @pl.when(s < NS - 1)
        def _():
            rb = s * OUT_T
            pltpu.sync_copy(acca_sh.at[pl.ds(rb, OUT_T)],
                            suma_hbm.at[pl.ds(base_row + rb, OUT_T)])
            pltpu.sync_copy(accb_sh.at[pl.ds(rb, OUT_T)],
                            sumb_hbm.at[pl.ds(base_row + rb, OUT_T)])
            if with_cnt:
                pltpu.sync_copy(cacc_sh.at[pl.ds(rb, OUT_T)],
                                cnt_hbm.at[pl.ds(base_row + rb, OUT_T)])

        @pl.when(s == NS - 1)
        def _():
            rb = (NS - 1) * OUT_T
            pltpu.sync_copy(acca_sh.at[pl.ds(rb, rem)],
                            suma_hbm.at[pl.ds(base_row + rb, rem)])
            pltpu.sync_copy(accb_sh.at[pl.ds(rb, rem)],
                            sumb_hbm.at[pl.ds(base_row + rb, rem)])
            if with_cnt:
                pltpu.sync_copy(cacc_sh.at[pl.ds(rb, rem)],
                                cnt_hbm.at[pl.ds(base_row + rb, rem)])

        if q == 0:
            lax.fori_loop(0, B, _zrow, 0)
            plsc.subcore_barrier()


def _make_sc_agg(with_cnt):
    body = functools.partial(_sc_agg_body, with_cnt=with_cnt)
    if with_cnt:
        def wrapped(xa, xb, src, dst, suma, sumb, cnt, *rest):
            body(xa, xb, src, dst, suma, sumb, cnt, *rest)
        out_type = (jax.ShapeDtypeStruct((N, DH), jnp.float32),
                    jax.ShapeDtypeStruct((N, DH), jnp.float32),
                    jax.ShapeDtypeStruct((N, 16), jnp.float32))
    else:
        def wrapped(xa, xb, src, dst, suma, sumb, *rest):
            body(xa, xb, src, dst, suma, sumb, None, *rest)
        out_type = (jax.ShapeDtypeStruct((N, DH), jnp.float32),
                    jax.ShapeDtypeStruct((N, DH), jnp.float32))
    return functools.partial(
        pl.kernel,
        mesh=plsc.VectorSubcoreMesh(core_axis_name="c", subcore_axis_name="s",
                                    num_cores=NC),
        compiler_params=pltpu.CompilerParams(needs_layout_passes=False,
                                             use_tc_tiling_on_sc=False),
        out_type=out_type,
        scratch_types=[
            pltpu.VMEM((PIECE,), jnp.int32),
            pltpu.VMEM((PIECE,), jnp.int32),
            pltpu.VMEM((CAP,), jnp.int32),
            pltpu.VMEM((CAP,), jnp.int32),
            pltpu.VMEM((B, DH), jnp.float32),
            pltpu.VMEM((B, DH), jnp.float32),
            pltpu.VMEM((B, DH), jnp.float32),
            pltpu.VMEM((B, DH), jnp.float32),
            pltpu.VMEM((B, 16), jnp.float32),
            pltpu.VMEM((B,), jnp.int32),
            pltpu.VMEM((B,), jnp.int32),
            pltpu.VMEM((B,), jnp.int32),
            pltpu.VMEM((B,), jnp.int32),
            pltpu.VMEM_SHARED((ACC, DH), jnp.float32),
            pltpu.VMEM_SHARED((ACC, DH), jnp.float32),
            pltpu.VMEM_SHARED((ACC, 16), jnp.float32),
            pltpu.SemaphoreType.DMA,
            pltpu.SemaphoreType.DMA,
            pltpu.SemaphoreType.DMA,
            pltpu.SemaphoreType.DMA,
        ],
    )(wrapped)


_sc_agg_cnt = _make_sc_agg(with_cnt=True)
_sc_agg_nocnt = _make_sc_agg(with_cnt=False)


RB = 2000


def _self_body(xa_ref, xb_ref, wr_ref, b_ref, o_ref):
    acc = lax.dot_general(xa_ref[...], wr_ref[:, 0:DH],
                          (((1,), (1,)), ((), ())),
                          preferred_element_type=jnp.float32)
    acc = acc + lax.dot_general(xb_ref[...], wr_ref[:, DH:D],
                                (((1,), (1,)), ((), ())),
                                preferred_element_type=jnp.float32)
    o_ref[...] = acc + b_ref[...]


def _self(xa, xb, Wr, b):
    return pl.pallas_call(
        _self_body,
        grid=(N // RB,),
        in_specs=[
            pl.BlockSpec((RB, DH), lambda i: (i, 0)),
            pl.BlockSpec((RB, DH), lambda i: (i, 0)),
            pl.BlockSpec((D, D), lambda i: (0, 0)),
            pl.BlockSpec((1, D), lambda i: (0, 0)),
        ],
        out_specs=pl.BlockSpec((RB, D), lambda i: (i, 0)),
        out_shape=jax.ShapeDtypeStruct((N, D), jnp.float32),
    )(xa, xb, Wr, b.reshape(1, D))


def _combine_body(suma_ref, sumb_ref, cnt_ref, t_ref, wl_ref,
                  oa_ref, ob_ref, *, relu, split_out):
    cnt = cnt_ref[:, 0:1]
    inv = 1.0 / jnp.maximum(cnt, 1.0)
    acc = lax.dot_general(suma_ref[...] * inv, wl_ref[:, 0:DH],
                          (((1,), (1,)), ((), ())),
                          preferred_element_type=jnp.float32)
    acc = acc + lax.dot_general(sumb_ref[...] * inv, wl_ref[:, DH:D],
                                (((1,), (1,)), ((), ())),
                                preferred_element_type=jnp.float32)
    acc = acc + t_ref[...]
    if relu:
        acc = jnp.maximum(acc, 0.0)
    if split_out:
        oa_ref[...] = acc[:, 0:DH]
        ob_ref[...] = acc[:, DH:D]
    else:
        oa_ref[...] = acc


def _combine(suma, sumb, cnt16, t, Wl, relu, split_out):
    if split_out:
        out_specs = (pl.BlockSpec((RB, DH), lambda i: (i, 0)),
                     pl.BlockSpec((RB, DH), lambda i: (i, 0)))
        out_shape = (jax.ShapeDtypeStruct((N, DH), jnp.float32),
                     jax.ShapeDtypeStruct((N, DH), jnp.float32))
    else:
        out_specs = pl.BlockSpec((RB, D), lambda i: (i, 0))
        out_shape = jax.ShapeDtypeStruct((N, D), jnp.float32)
    body = functools.partial(_combine_body, relu=relu, split_out=split_out)
    if not split_out:
        body_wrapped = lambda *refs: body(*refs[:-1], refs[-1], None)

        def body_wrapped(suma_ref, sumb_ref, cnt_ref, t_ref, wl_ref, o_ref):
            body(suma_ref, sumb_ref, cnt_ref, t_ref, wl_ref, o_ref, None)
    else:
        body_wrapped = body
    return pl.pallas_call(
        body_wrapped,
        grid=(N // RB,),
        in_specs=[
            pl.BlockSpec((RB, DH), lambda i: (i, 0)),
            pl.BlockSpec((RB, DH), lambda i: (i, 0)),
            pl.BlockSpec((RB, 16), lambda i: (i, 0)),
            pl.BlockSpec((RB, D), lambda i: (i, 0)),
            pl.BlockSpec((D, D), lambda i: (0, 0)),
        ],
        out_specs=out_specs,
        out_shape=out_shape,
    )(suma, sumb, cnt16, t, Wl)


def kernel(x, edge_index, W1l, b1l, W1r, W2l, b2l, W2r):
    ei = edge_index.astype(jnp.int32)
    src = ei[0]
    dst = ei[1]
    xa = x[:, 0:DH]
    xb = x[:, DH:D]
    s1a, s1b, c1 = _sc_agg_cnt(xa, xb, src, dst)
    t1 = _self(xa, xb, W1r, b1l)
    ha, hb = _combine(s1a, s1b, c1, t1, W1l, relu=True, split_out=True)
    s2a, s2b = _sc_agg_nocnt(ha, hb, src, dst)
    t2 = _self(ha, hb, W2r, b2l)
    out = _combine(s2a, s2b, c1, t2, W2l, relu=False, split_out=False)
    return out
